# baseline (speedup 1.0000x reference)
import jax
import jax.numpy as jnp
from jax.experimental import pallas as pl
from jax.experimental.pallas import tpu as pltpu

_SMOOTH = 1.0
_HT = 256
_NS = 512 // _HT
_NC = 19


def _dice_tc_body(pred_ref, tgt_ref, out_ref, nom_acc, isum_acc, tsum_acc):
    s = pl.program_id(1)

    @pl.when(s == 0)
    def _init():
        nom_acc[...] = jnp.zeros_like(nom_acc)
        isum_acc[...] = jnp.zeros_like(isum_acc)
        tsum_acc[...] = jnp.zeros_like(tsum_acc)

    zero = jnp.zeros((8, 512), jnp.float32)
    for c in range(_NC):
        def h_body(h, carry, c=c):
            nom_v, isum_v, tsum_v = carry
            pred_p = pred_ref[0, c, pl.ds(h * 8, 8), :]
            tgt_p = tgt_ref[0, pl.ds(h * 8, 8), :]
            maskf = jnp.where(tgt_p == c, 1.0, 0.0)
            return (nom_v + pred_p * maskf,
                    isum_v + pred_p,
                    tsum_v + maskf)

        nom_v, isum_v, tsum_v = jax.lax.fori_loop(
            0, _HT // 8, h_body, (zero, zero, zero))
        nom_acc[c] += nom_v
        isum_acc[c] += isum_v
        tsum_acc[c] += tsum_v

    @pl.when(s == _NS - 1)
    def _finish():
        nom = jnp.sum(nom_acc[...], axis=(1, 2))
        isum = jnp.sum(isum_acc[...], axis=(1, 2))
        tsum = jnp.sum(tsum_acc[...], axis=(1, 2))
        frac = (2.0 * nom + _SMOOTH) / (isum + tsum + _SMOOTH)
        loss = -jnp.sum(frac) / _NC
        out_ref[0, 0, :] = jnp.full((128,), loss, dtype=jnp.float32)


def kernel(predictions, targets):
    out = pl.pallas_call(
        _dice_tc_body,
        grid=(8, _NS),
        in_specs=[
            pl.BlockSpec((1, _NC, _HT, 512), lambda b, s: (b, 0, s, 0)),
            pl.BlockSpec((1, _HT, 512), lambda b, s: (b, s, 0)),
        ],
        out_specs=pl.BlockSpec((1, 1, 128), lambda b, s: (b, 0, 0)),
        out_shape=jax.ShapeDtypeStruct((8, 1, 128), jnp.float32),
        scratch_shapes=[
            pltpu.VMEM((_NC, 8, 512), jnp.float32),
            pltpu.VMEM((_NC, 8, 512), jnp.float32),
            pltpu.VMEM((_NC, 8, 512), jnp.float32),
        ],
    )(predictions, targets)
    return out[:, 0, 0]


# speedup vs baseline: 1.2160x; 1.2160x over previous
import jax
import jax.numpy as jnp
from jax.experimental import pallas as pl
from jax.experimental.pallas import tpu as pltpu

_SMOOTH = 1.0
_NC = 19


def _dice_tc_body(pred_ref, tgt_ref, out_ref):
    zero = jnp.zeros((8, 512), jnp.float32)
    total = 0.0
    for c in range(_NC):
        def h_body(h, carry, c=c):
            nom_v, isum_v, tsum_v = carry
            pred_p = pred_ref[0, c, pl.ds(h * 8, 8), :]
            tgt_p = tgt_ref[0, pl.ds(h * 8, 8), :]
            maskf = jnp.where(tgt_p == c, 1.0, 0.0)
            return (nom_v + pred_p * maskf,
                    isum_v + pred_p,
                    tsum_v + maskf)

        nom_v, isum_v, tsum_v = jax.lax.fori_loop(
            0, 64, h_body, (zero, zero, zero), unroll=8)
        nom = jnp.sum(nom_v)
        isum = jnp.sum(isum_v)
        tsum = jnp.sum(tsum_v)
        total += (2.0 * nom + _SMOOTH) / (isum + tsum + _SMOOTH)

    out_ref[0, 0, :] = jnp.full((128,), -total / _NC, dtype=jnp.float32)


def kernel(predictions, targets):
    out = pl.pallas_call(
        _dice_tc_body,
        grid=(8,),
        in_specs=[
            pl.BlockSpec((1, _NC, 512, 512), lambda b: (b, 0, 0, 0)),
            pl.BlockSpec((1, 512, 512), lambda b: (b, 0, 0)),
        ],
        out_specs=pl.BlockSpec((1, 1, 128), lambda b: (b, 0, 0)),
        out_shape=jax.ShapeDtypeStruct((8, 1, 128), jnp.float32),
    )(predictions, targets)
    return out[:, 0, 0]
